# Initial kernel scaffold; baseline (speedup 1.0000x reference)
import jax
import jax.numpy as jnp
from jax.experimental import pallas as pl


def kernel(x, log_tau, log_beta, log_gamma):
    raise NotImplementedError("write your pallas kernel here")


# speedup vs baseline: 18.9320x; 18.9320x over previous
import functools
import math

import jax
import jax.numpy as jnp
from jax.experimental import pallas as pl

_K = 64
_EPS = 1e-05
_EPS_VAR = 1e-04
_C = math.sqrt(2.0 / math.pi)
_ROWS = 256
_NIT = 20


def _gelu(x):
    return 0.5 * x * (1.0 + jnp.tanh(_C * (x + 0.044715 * x * x * x)))


def _stats_kernel(x_ref, sums_ref):
    x = x_ref[...]
    o = _gelu(x)
    p = jnp.concatenate(
        [jnp.sum(x, axis=0)[None, :],
         jnp.sum(x * x, axis=0)[None, :],
         jnp.sum(o, axis=0)[None, :],
         jnp.sum(o * o, axis=0)[None, :],
         jnp.zeros((4, x.shape[1]), jnp.float32)],
        axis=0)

    @pl.when(pl.program_id(0) == 0)
    def _():
        sums_ref[...] = p

    @pl.when(pl.program_id(0) != 0)
    def _():
        sums_ref[...] += p


def _kth_threshold(a):
    hi = jnp.max(a, axis=1, keepdims=True)
    lo = jnp.zeros_like(hi)

    def body(_, carry):
        lo, hi = carry
        mid = 0.5 * (lo + hi)
        cnt = jnp.sum((a > mid).astype(jnp.float32), axis=1, keepdims=True)
        ge = cnt >= _K
        return jnp.where(ge, mid, lo), jnp.where(ge, hi, mid)

    lo, _ = jax.lax.fori_loop(0, _NIT, body, (lo, hi))
    return lo


def _apply_kernel(inv_n, stats_ref, x_ref, out_ref):
    mean_in = stats_ref[0:1, :] * inv_n
    sq_in = stats_ref[1:2, :] * inv_n
    mean_out = stats_ref[2:3, :] * inv_n
    sq_out = stats_ref[3:4, :] * inv_n
    tau = stats_ref[4:5, 0:1]
    beta = stats_ref[5:6, 0:1]
    gamma = stats_ref[6:7, 0:1]
    var_in = jnp.clip(sq_in - mean_in * mean_in, _EPS_VAR, None)
    rstd_in = 1.0 / (jnp.sqrt(var_in) + _EPS)
    var_out = jnp.clip(sq_out - mean_out * mean_out, _EPS_VAR, None)
    rstd_out = 1.0 / (jnp.sqrt(var_out) + _EPS)
    ema_n = mean_out / jnp.maximum(
        jnp.sqrt(jnp.sum(mean_out * mean_out)), 1e-12)

    x = x_ref[...]
    o = _gelu(x)
    z_in = (x - mean_in) * rstd_in
    z_out = (o - mean_out) * rstd_out
    a_in = jnp.abs(z_in)
    a_out = jnp.abs(z_out)
    inter = (a_in > _kth_threshold(a_in)) & (a_out > _kth_threshold(a_out))
    gate = jnp.clip(1.0 + beta * jnp.tanh(gamma * z_in), 0.1, 8.0)
    gate_vec = jnp.where(inter, gate, 1.0)
    norm = jnp.maximum(jnp.sqrt(jnp.sum(o * o, axis=1, keepdims=True)), 1e-12)
    cos = jnp.clip(jnp.sum(o * ema_n, axis=1, keepdims=True) / norm,
                   -1.0, 1.0)
    out_ref[...] = o * gate_vec * jnp.exp(-tau * cos)


def kernel(x, log_tau, log_beta, log_gamma):
    B, T, D = x.shape
    n = B * T
    xf = x.reshape(n, D)
    sums = pl.pallas_call(
        _stats_kernel,
        grid=(n // _ROWS,),
        in_specs=[pl.BlockSpec((_ROWS, D), lambda i: (i, 0))],
        out_specs=pl.BlockSpec((8, D), lambda i: (0, 0)),
        out_shape=jax.ShapeDtypeStruct((8, D), jnp.float32),
    )(xf)
    tau = jnp.exp(log_tau).astype(jnp.float32)
    beta = jax.nn.softplus(log_beta).astype(jnp.float32)
    gamma = jax.nn.softplus(log_gamma).astype(jnp.float32)
    stats = jnp.concatenate(
        [sums[0:4],
         jnp.broadcast_to(tau, (1, D)),
         jnp.broadcast_to(beta, (1, D)),
         jnp.broadcast_to(gamma, (1, D)),
         jnp.zeros((1, D), jnp.float32)], axis=0)
    out = pl.pallas_call(
        functools.partial(_apply_kernel, 1.0 / n),
        grid=(n // _ROWS,),
        in_specs=[pl.BlockSpec((8, D), lambda i: (0, 0)),
                  pl.BlockSpec((_ROWS, D), lambda i: (i, 0))],
        out_specs=pl.BlockSpec((_ROWS, D), lambda i: (i, 0)),
        out_shape=jax.ShapeDtypeStruct((n, D), jnp.float32),
    )(stats, xf)
    return out.reshape(B, T, D)


# speedup vs baseline: 21.6561x; 1.1439x over previous
import functools
import math

import jax
import jax.numpy as jnp
from jax.experimental import pallas as pl

_K = 64
_EPS = 1e-05
_EPS_VAR = 1e-04
_C = math.sqrt(2.0 / math.pi)
_ROWS = 256
_NIT = 16


def _gelu(x):
    return 0.5 * x * (1.0 + jnp.tanh(_C * (x + 0.044715 * x * x * x)))


def _stats_kernel(x_ref, sums_ref):
    x = x_ref[...]
    o = _gelu(x)
    p = jnp.concatenate(
        [jnp.sum(x, axis=0)[None, :],
         jnp.sum(x * x, axis=0)[None, :],
         jnp.sum(o, axis=0)[None, :],
         jnp.sum(o * o, axis=0)[None, :],
         jnp.zeros((4, x.shape[1]), jnp.float32)],
        axis=0)

    @pl.when(pl.program_id(0) == 0)
    def _():
        sums_ref[...] = p

    @pl.when(pl.program_id(0) != 0)
    def _():
        sums_ref[...] += p


def _kth_threshold(a):
    hi = jnp.max(a, axis=1, keepdims=True)
    lo = jnp.zeros_like(hi)

    def body(_, carry):
        lo, hi = carry
        mid = 0.5 * (lo + hi)
        cnt = jnp.sum((a > mid).astype(jnp.float32), axis=1, keepdims=True)
        ge = cnt >= _K
        return jnp.where(ge, mid, lo), jnp.where(ge, hi, mid)

    lo, _ = jax.lax.fori_loop(0, _NIT, body, (lo, hi))
    return lo


def _apply_kernel(inv_n, stats_ref, x_ref, out_ref):
    mean_in = stats_ref[0:1, :] * inv_n
    sq_in = stats_ref[1:2, :] * inv_n
    mean_out = stats_ref[2:3, :] * inv_n
    sq_out = stats_ref[3:4, :] * inv_n
    tau = stats_ref[4:5, 0:1]
    beta = stats_ref[5:6, 0:1]
    gamma = stats_ref[6:7, 0:1]
    var_in = jnp.clip(sq_in - mean_in * mean_in, _EPS_VAR, None)
    rstd_in = 1.0 / (jnp.sqrt(var_in) + _EPS)
    var_out = jnp.clip(sq_out - mean_out * mean_out, _EPS_VAR, None)
    rstd_out = 1.0 / (jnp.sqrt(var_out) + _EPS)
    ema_n = mean_out / jnp.maximum(
        jnp.sqrt(jnp.sum(mean_out * mean_out)), 1e-12)

    x = x_ref[...]
    o = _gelu(x)
    z_in = (x - mean_in) * rstd_in
    z_out = (o - mean_out) * rstd_out
    a_in = jnp.abs(z_in)
    a_out = jnp.abs(z_out)
    inter = (a_in > _kth_threshold(a_in)) & (a_out > _kth_threshold(a_out))
    gate = jnp.clip(1.0 + beta * jnp.tanh(gamma * z_in), 0.1, 8.0)
    gate_vec = jnp.where(inter, gate, 1.0)
    norm = jnp.maximum(jnp.sqrt(jnp.sum(o * o, axis=1, keepdims=True)), 1e-12)
    cos = jnp.clip(jnp.sum(o * ema_n, axis=1, keepdims=True) / norm,
                   -1.0, 1.0)
    out_ref[...] = o * gate_vec * jnp.exp(-tau * cos)


def kernel(x, log_tau, log_beta, log_gamma):
    B, T, D = x.shape
    n = B * T
    xf = x.reshape(n, D)
    sums = pl.pallas_call(
        _stats_kernel,
        grid=(n // _ROWS,),
        in_specs=[pl.BlockSpec((_ROWS, D), lambda i: (i, 0))],
        out_specs=pl.BlockSpec((8, D), lambda i: (0, 0)),
        out_shape=jax.ShapeDtypeStruct((8, D), jnp.float32),
    )(xf)
    tau = jnp.exp(log_tau).astype(jnp.float32)
    beta = jax.nn.softplus(log_beta).astype(jnp.float32)
    gamma = jax.nn.softplus(log_gamma).astype(jnp.float32)
    stats = jnp.concatenate(
        [sums[0:4],
         jnp.broadcast_to(tau, (1, D)),
         jnp.broadcast_to(beta, (1, D)),
         jnp.broadcast_to(gamma, (1, D)),
         jnp.zeros((1, D), jnp.float32)], axis=0)
    out = pl.pallas_call(
        functools.partial(_apply_kernel, 1.0 / n),
        grid=(n // _ROWS,),
        in_specs=[pl.BlockSpec((8, D), lambda i: (0, 0)),
                  pl.BlockSpec((_ROWS, D), lambda i: (i, 0))],
        out_specs=pl.BlockSpec((_ROWS, D), lambda i: (i, 0)),
        out_shape=jax.ShapeDtypeStruct((n, D), jnp.float32),
    )(stats, xf)
    return out.reshape(B, T, D)


# speedup vs baseline: 26.1388x; 1.2070x over previous
import functools
import math

import jax
import jax.numpy as jnp
from jax.experimental import pallas as pl

_K = 64
_EPS = 1e-05
_EPS_VAR = 1e-04
_C = math.sqrt(2.0 / math.pi)
_ROWS = 256
_NIT = 14


def _gelu(x):
    return 0.5 * x * (1.0 + jnp.tanh(_C * (x + 0.044715 * x * x * x)))


def _stats_kernel(x_ref, sums_ref):
    x = x_ref[...]
    o = _gelu(x)
    p = jnp.concatenate(
        [jnp.sum(x, axis=0)[None, :],
         jnp.sum(x * x, axis=0)[None, :],
         jnp.sum(o, axis=0)[None, :],
         jnp.sum(o * o, axis=0)[None, :],
         jnp.zeros((4, x.shape[1]), jnp.float32)],
        axis=0)

    @pl.when(pl.program_id(0) == 0)
    def _():
        sums_ref[...] = p

    @pl.when(pl.program_id(0) != 0)
    def _():
        sums_ref[...] += p


def _kth_threshold2(a, b):
    hi_a = jnp.max(a, axis=1, keepdims=True)
    lo_a = jnp.zeros_like(hi_a)
    hi_b = jnp.max(b, axis=1, keepdims=True)
    lo_b = jnp.zeros_like(hi_b)

    def body(_, carry):
        lo_a, hi_a, lo_b, hi_b = carry
        mid_a = 0.5 * (lo_a + hi_a)
        mid_b = 0.5 * (lo_b + hi_b)
        cnt_a = jnp.sum((a > mid_a).astype(jnp.float32), axis=1,
                        keepdims=True)
        cnt_b = jnp.sum((b > mid_b).astype(jnp.float32), axis=1,
                        keepdims=True)
        ge_a = cnt_a >= _K
        ge_b = cnt_b >= _K
        return (jnp.where(ge_a, mid_a, lo_a), jnp.where(ge_a, hi_a, mid_a),
                jnp.where(ge_b, mid_b, lo_b), jnp.where(ge_b, hi_b, mid_b))

    lo_a, _, lo_b, _ = jax.lax.fori_loop(
        0, _NIT, body, (lo_a, hi_a, lo_b, hi_b))
    return lo_a, lo_b


def _apply_kernel(inv_n, stats_ref, x_ref, out_ref):
    mean_in = stats_ref[0:1, :] * inv_n
    sq_in = stats_ref[1:2, :] * inv_n
    mean_out = stats_ref[2:3, :] * inv_n
    sq_out = stats_ref[3:4, :] * inv_n
    tau = stats_ref[4:5, 0:1]
    beta = stats_ref[5:6, 0:1]
    gamma = stats_ref[6:7, 0:1]
    var_in = jnp.clip(sq_in - mean_in * mean_in, _EPS_VAR, None)
    rstd_in = 1.0 / (jnp.sqrt(var_in) + _EPS)
    var_out = jnp.clip(sq_out - mean_out * mean_out, _EPS_VAR, None)
    rstd_out = 1.0 / (jnp.sqrt(var_out) + _EPS)
    ema_n = mean_out / jnp.maximum(
        jnp.sqrt(jnp.sum(mean_out * mean_out)), 1e-12)

    x = x_ref[...]
    o = _gelu(x)
    z_in = (x - mean_in) * rstd_in
    z_out = (o - mean_out) * rstd_out
    a_in = jnp.abs(z_in)
    a_out = jnp.abs(z_out)
    thr_in, thr_out = _kth_threshold2(a_in, a_out)
    inter = (a_in > thr_in) & (a_out > thr_out)
    gate = jnp.clip(1.0 + beta * jnp.tanh(gamma * z_in), 0.1, 8.0)
    gate_vec = jnp.where(inter, gate, 1.0)
    norm = jnp.maximum(jnp.sqrt(jnp.sum(o * o, axis=1, keepdims=True)), 1e-12)
    cos = jnp.clip(jnp.sum(o * ema_n, axis=1, keepdims=True) / norm,
                   -1.0, 1.0)
    out_ref[...] = o * gate_vec * jnp.exp(-tau * cos)


def kernel(x, log_tau, log_beta, log_gamma):
    B, T, D = x.shape
    n = B * T
    xf = x.reshape(n, D)
    sums = pl.pallas_call(
        _stats_kernel,
        grid=(n // _ROWS,),
        in_specs=[pl.BlockSpec((_ROWS, D), lambda i: (i, 0))],
        out_specs=pl.BlockSpec((8, D), lambda i: (0, 0)),
        out_shape=jax.ShapeDtypeStruct((8, D), jnp.float32),
    )(xf)
    tau = jnp.exp(log_tau).astype(jnp.float32)
    beta = jax.nn.softplus(log_beta).astype(jnp.float32)
    gamma = jax.nn.softplus(log_gamma).astype(jnp.float32)
    stats = jnp.concatenate(
        [sums[0:4],
         jnp.broadcast_to(tau, (1, D)),
         jnp.broadcast_to(beta, (1, D)),
         jnp.broadcast_to(gamma, (1, D)),
         jnp.zeros((1, D), jnp.float32)], axis=0)
    out = pl.pallas_call(
        functools.partial(_apply_kernel, 1.0 / n),
        grid=(n // _ROWS,),
        in_specs=[pl.BlockSpec((8, D), lambda i: (0, 0)),
                  pl.BlockSpec((_ROWS, D), lambda i: (i, 0))],
        out_specs=pl.BlockSpec((_ROWS, D), lambda i: (i, 0)),
        out_shape=jax.ShapeDtypeStruct((n, D), jnp.float32),
    )(stats, xf)
    return out.reshape(B, T, D)
